# baseline (speedup 1.0000x reference)
import jax
import jax.numpy as jnp
from jax.experimental import pallas as pl
from jax.experimental.pallas import tpu as pltpu

_B, _N, _D, _O = 32, 512, 128, 128


def _fused_attention_kernel(x_ref, W_ref, asrc_ref, adst_ref, bias_ref, o_ref):
    f32 = jnp.float32
    x = x_ref[0]
    W = W_ref[...]
    H = jnp.dot(x, W, preferred_element_type=f32)

    a_src = asrc_ref[...]
    a_dst = adst_ref[...]
    d_col = jax.lax.dot_general(H, a_dst, (((1,), (1,)), ((), ())),
                                preferred_element_type=f32)
    s_row = jax.lax.dot_general(a_src, H, (((1,), (1,)), ((), ())),
                                preferred_element_type=f32)

    s_max = jnp.max(s_row, axis=1, keepdims=True)
    dm = d_col + s_max
    m_col = jnp.maximum(dm, 0.2 * dm)

    ones_col = jnp.ones((_N, 1), dtype=f32)
    ones_row = jnp.ones((1, _N), dtype=f32)
    rhs = jnp.concatenate([ones_row, s_row], axis=0)
    lhs_a = jnp.concatenate([d_col - m_col, ones_col], axis=1)
    lhs_b = jnp.concatenate([0.2 * d_col - m_col,
                             0.2 * ones_col], axis=1)
    A = jnp.dot(lhs_a, rhs, preferred_element_type=f32)
    Bm = jnp.dot(lhs_b, rhs, preferred_element_type=f32)
    p = jnp.exp(jnp.maximum(A, Bm))

    denom = jnp.dot(p, ones_col, preferred_element_type=f32)
    num = jnp.dot(p, H, preferred_element_type=f32)
    out = num / denom + bias_ref[...]
    o_ref[0] = jnp.where(out > 0, out, jnp.exp(out) - 1.0)


def kernel(x, W, a_src, a_dst, bias):
    grid = (_B,)
    out = pl.pallas_call(
        _fused_attention_kernel,
        grid=grid,
        in_specs=[
            pl.BlockSpec((1, _N, _D), lambda b: (b, 0, 0)),
            pl.BlockSpec((_D, _O), lambda b: (0, 0)),
            pl.BlockSpec((1, _O), lambda b: (0, 0)),
            pl.BlockSpec((1, _O), lambda b: (0, 0)),
            pl.BlockSpec((1, _O), lambda b: (0, 0)),
        ],
        out_specs=pl.BlockSpec((1, _N, _O), lambda b: (b, 0, 0)),
        out_shape=jax.ShapeDtypeStruct((_B, _N, _O), jnp.float32),
    )(x, W, a_src.reshape(1, _O), a_dst.reshape(1, _O), bias.reshape(1, _O))
    return out


# speedup vs baseline: 1.1148x; 1.1148x over previous
import jax
import jax.numpy as jnp
from jax.experimental import pallas as pl
from jax.experimental.pallas import tpu as pltpu

_B, _N, _D, _O = 32, 512, 128, 128


def _fused_attention_kernel(x_ref, W_ref, asrc_ref, adst_ref, bias_ref, o_ref):
    f32 = jnp.float32
    x = x_ref[0]
    W = W_ref[...]
    H = jnp.dot(x, W, preferred_element_type=f32)

    a_src = asrc_ref[...]
    a_dst = adst_ref[...]
    d_col = jax.lax.dot_general(H, a_dst, (((1,), (1,)), ((), ())),
                                preferred_element_type=f32)
    s_row = jax.lax.dot_general(a_src, H, (((1,), (1,)), ((), ())),
                                preferred_element_type=f32)

    s_max = jnp.max(s_row, axis=1, keepdims=True)
    dm = d_col + s_max
    m_col = jnp.maximum(dm, 0.2 * dm)

    da = d_col - m_col
    db = 0.2 * d_col - m_col
    sb = 0.2 * s_row
    t = jnp.maximum(da + s_row, db + sb)
    p = jnp.exp(t)

    ones_col = jnp.ones((_N, 1), dtype=f32)
    denom = jnp.dot(p, ones_col, preferred_element_type=f32)
    num = jnp.dot(p, H, preferred_element_type=f32)
    out = num / denom + bias_ref[...]
    o_ref[0] = jnp.where(out > 0, out, jnp.exp(out) - 1.0)


def kernel(x, W, a_src, a_dst, bias):
    grid = (_B,)
    out = pl.pallas_call(
        _fused_attention_kernel,
        grid=grid,
        in_specs=[
            pl.BlockSpec((1, _N, _D), lambda b: (b, 0, 0)),
            pl.BlockSpec((_D, _O), lambda b: (0, 0)),
            pl.BlockSpec((1, _O), lambda b: (0, 0)),
            pl.BlockSpec((1, _O), lambda b: (0, 0)),
            pl.BlockSpec((1, _O), lambda b: (0, 0)),
        ],
        out_specs=pl.BlockSpec((1, _N, _O), lambda b: (b, 0, 0)),
        out_shape=jax.ShapeDtypeStruct((_B, _N, _O), jnp.float32),
    )(x, W, a_src.reshape(1, _O), a_dst.reshape(1, _O), bias.reshape(1, _O))
    return out


# speedup vs baseline: 1.1449x; 1.0269x over previous
import jax
import jax.numpy as jnp
from jax.experimental import pallas as pl
from jax.experimental.pallas import tpu as pltpu

_B, _N, _D, _O = 32, 512, 128, 128
_RB = 128
_LOG2E = 1.4426950408889634


def _fused_attention_kernel(x_ref, W_ref, asrc_ref, adst_ref, bias_ref, o_ref):
    f32 = jnp.float32
    x = x_ref[0]
    W = W_ref[...]
    H = jnp.dot(x, W, preferred_element_type=f32)

    a_src = asrc_ref[...]
    a_dst = adst_ref[...]
    d_col = jax.lax.dot_general(H, a_dst, (((1,), (1,)), ((), ())),
                                preferred_element_type=f32)
    s_row = jax.lax.dot_general(a_src, H, (((1,), (1,)), ((), ())),
                                preferred_element_type=f32)

    s_max = jnp.max(s_row, axis=1, keepdims=True)
    dm = d_col + s_max
    m_col = jnp.maximum(dm, 0.2 * dm)

    da = _LOG2E * (d_col - m_col)
    db = _LOG2E * (0.2 * d_col - m_col)
    sa = _LOG2E * s_row
    sb = (0.2 * _LOG2E) * s_row

    ones_col = jnp.ones((_N, 1), dtype=f32)
    bias_v = bias_ref[...]
    for r in range(_N // _RB):
        rs = slice(r * _RB, (r + 1) * _RB)
        t = jnp.maximum(da[rs] + sa, db[rs] + sb)
        p = jnp.exp2(t)
        denom = jnp.dot(p, ones_col, preferred_element_type=f32)
        num = jnp.dot(p, H, preferred_element_type=f32)
        out = num / denom + bias_v
        o_ref[0, rs] = jnp.where(out > 0, out, jnp.exp(out) - 1.0)


def kernel(x, W, a_src, a_dst, bias):
    grid = (_B,)
    out = pl.pallas_call(
        _fused_attention_kernel,
        grid=grid,
        in_specs=[
            pl.BlockSpec((1, _N, _D), lambda b: (b, 0, 0)),
            pl.BlockSpec((_D, _O), lambda b: (0, 0)),
            pl.BlockSpec((1, _O), lambda b: (0, 0)),
            pl.BlockSpec((1, _O), lambda b: (0, 0)),
            pl.BlockSpec((1, _O), lambda b: (0, 0)),
        ],
        out_specs=pl.BlockSpec((1, _N, _O), lambda b: (b, 0, 0)),
        out_shape=jax.ShapeDtypeStruct((_B, _N, _O), jnp.float32),
    )(x, W, a_src.reshape(1, _O), a_dst.reshape(1, _O), bias.reshape(1, _O))
    return out


# speedup vs baseline: 1.1454x; 1.0005x over previous
import jax
import jax.numpy as jnp
from jax.experimental import pallas as pl
from jax.experimental.pallas import tpu as pltpu

_B, _N, _D, _O = 32, 512, 128, 128
_RB = 128
_LOG2E = 1.4426950408889634


def _fused_attention_kernel(x_ref, W_ref, asrc_ref, adst_ref, bias_ref, o_ref):
    f32 = jnp.float32
    x = x_ref[0]
    W = W_ref[...]
    H = jnp.dot(x, W, preferred_element_type=f32)

    a_src = asrc_ref[...]
    a_dst = adst_ref[...]
    d_col = jax.lax.dot_general(H, a_dst, (((1,), (1,)), ((), ())),
                                preferred_element_type=f32)
    s_row = jax.lax.dot_general(a_src, H, (((1,), (1,)), ((), ())),
                                preferred_element_type=f32)

    s_max = jnp.max(s_row, axis=1, keepdims=True)
    dm = d_col + s_max
    m_col = jnp.maximum(dm, 0.2 * dm)

    da = _LOG2E * (d_col - m_col)
    db = _LOG2E * (0.2 * d_col - m_col)
    sa = _LOG2E * s_row
    sb = (0.2 * _LOG2E) * s_row

    ones_col = jnp.ones((_N, 1), dtype=f32)
    bias_v = bias_ref[...]
    for r in range(_N // _RB):
        rs = slice(r * _RB, (r + 1) * _RB)
        t = jnp.maximum(da[rs] + sa, db[rs] + sb)
        p = jnp.exp2(t)
        denom = jnp.dot(p, ones_col, preferred_element_type=f32)
        num = jnp.dot(p, H, preferred_element_type=f32)
        out = num / denom + bias_v
        o_ref[0, rs] = jnp.where(out > 0, out, jnp.exp(out) - 1.0)


def kernel(x, W, a_src, a_dst, bias):
    grid = (_B,)
    out = pl.pallas_call(
        _fused_attention_kernel,
        grid=grid,
        in_specs=[
            pl.BlockSpec((1, _N, _D), lambda b: (b, 0, 0)),
            pl.BlockSpec((_D, _O), lambda b: (0, 0)),
            pl.BlockSpec((1, _O), lambda b: (0, 0)),
            pl.BlockSpec((1, _O), lambda b: (0, 0)),
            pl.BlockSpec((1, _O), lambda b: (0, 0)),
        ],
        out_specs=pl.BlockSpec((1, _N, _O), lambda b: (b, 0, 0)),
        out_shape=jax.ShapeDtypeStruct((_B, _N, _O), jnp.float32),
        compiler_params=pltpu.CompilerParams(
            dimension_semantics=("parallel",)),
    )(x, W, a_src.reshape(1, _O), a_dst.reshape(1, _O), bias.reshape(1, _O))
    return out


# speedup vs baseline: 1.4652x; 1.2791x over previous
import jax
import jax.numpy as jnp
from jax.experimental import pallas as pl
from jax.experimental.pallas import tpu as pltpu

_B, _N, _D, _O = 32, 512, 128, 128
_S = 4
_RB = 128
_LOG2E = 1.4426950408889634


def _fused_attention_kernel(x_ref, W_ref, asrc_ref, adst_ref, bias_ref, o_ref):
    f32 = jnp.float32
    x = x_ref[...].reshape(_S * _N, _D)
    W = W_ref[...]
    H_all = jnp.dot(x, W, preferred_element_type=f32)

    a_src = asrc_ref[...]
    a_dst = adst_ref[...]
    d_all = jax.lax.dot_general(H_all, a_dst, (((1,), (1,)), ((), ())),
                                preferred_element_type=f32)

    ones_col = jnp.ones((_N, 1), dtype=f32)
    bias_v = bias_ref[...]
    for i in range(_S):
        ss = slice(i * _N, (i + 1) * _N)
        H = H_all[ss]
        d_col = d_all[ss]
        s_row = jax.lax.dot_general(a_src, H, (((1,), (1,)), ((), ())),
                                    preferred_element_type=f32)

        s_max = jnp.max(s_row, axis=1, keepdims=True)
        dm = d_col + s_max
        m_col = jnp.maximum(dm, 0.2 * dm)

        da = _LOG2E * (d_col - m_col)
        db = _LOG2E * (0.2 * d_col - m_col)
        sa = _LOG2E * s_row
        sb = 0.2 * sa

        for r in range(_N // _RB):
            rs = slice(r * _RB, (r + 1) * _RB)
            t = jnp.maximum(da[rs] + sa, db[rs] + sb)
            p = jnp.exp2(t)
            denom = jnp.dot(p, ones_col, preferred_element_type=f32)
            num = jnp.dot(p, H, preferred_element_type=f32)
            out = num / denom + bias_v
            o_ref[i, rs] = jnp.where(out > 0, out, jnp.exp(out) - 1.0)


def kernel(x, W, a_src, a_dst, bias):
    grid = (_B // _S,)
    out = pl.pallas_call(
        _fused_attention_kernel,
        grid=grid,
        in_specs=[
            pl.BlockSpec((_S, _N, _D), lambda b: (b, 0, 0)),
            pl.BlockSpec((_D, _O), lambda b: (0, 0)),
            pl.BlockSpec((1, _O), lambda b: (0, 0)),
            pl.BlockSpec((1, _O), lambda b: (0, 0)),
            pl.BlockSpec((1, _O), lambda b: (0, 0)),
        ],
        out_specs=pl.BlockSpec((_S, _N, _O), lambda b: (b, 0, 0)),
        out_shape=jax.ShapeDtypeStruct((_B, _N, _O), jnp.float32),
        compiler_params=pltpu.CompilerParams(
            dimension_semantics=("parallel",)),
    )(x, W, a_src.reshape(1, _O), a_dst.reshape(1, _O), bias.reshape(1, _O))
    return out


# speedup vs baseline: 1.5466x; 1.0556x over previous
import jax
import jax.numpy as jnp
from jax.experimental import pallas as pl
from jax.experimental.pallas import tpu as pltpu

_B, _N, _D, _O = 32, 512, 128, 128
_S = 8
_RB = 128
_LOG2E = 1.4426950408889634


def _fused_attention_kernel(x_ref, W_ref, asrc_ref, adst_ref, bias_ref, o_ref):
    f32 = jnp.float32
    x = x_ref[...].reshape(_S * _N, _D)
    W = W_ref[...]
    H_all = jnp.dot(x, W, preferred_element_type=f32)

    a_src = asrc_ref[...]
    a_dst = adst_ref[...]
    d_all = jax.lax.dot_general(H_all, a_dst, (((1,), (1,)), ((), ())),
                                preferred_element_type=f32)

    ones_col = jnp.ones((_N, 1), dtype=f32)
    bias_v = bias_ref[...]
    for i in range(_S):
        ss = slice(i * _N, (i + 1) * _N)
        H = H_all[ss]
        d_col = d_all[ss]
        s_row = jax.lax.dot_general(a_src, H, (((1,), (1,)), ((), ())),
                                    preferred_element_type=f32)

        s_max = jnp.max(s_row, axis=1, keepdims=True)
        dm = d_col + s_max
        m_col = jnp.maximum(dm, 0.2 * dm)

        da = _LOG2E * (d_col - m_col)
        db = _LOG2E * (0.2 * d_col - m_col)
        sa = _LOG2E * s_row
        sb = 0.2 * sa

        for r in range(_N // _RB):
            rs = slice(r * _RB, (r + 1) * _RB)
            t = jnp.maximum(da[rs] + sa, db[rs] + sb)
            p = jnp.exp2(t)
            denom = jnp.dot(p, ones_col, preferred_element_type=f32)
            num = jnp.dot(p, H, preferred_element_type=f32)
            out = num / denom + bias_v
            o_ref[i, rs] = jnp.where(out > 0, out, jnp.exp(out) - 1.0)


def kernel(x, W, a_src, a_dst, bias):
    grid = (_B // _S,)
    out = pl.pallas_call(
        _fused_attention_kernel,
        grid=grid,
        in_specs=[
            pl.BlockSpec((_S, _N, _D), lambda b: (b, 0, 0)),
            pl.BlockSpec((_D, _O), lambda b: (0, 0)),
            pl.BlockSpec((1, _O), lambda b: (0, 0)),
            pl.BlockSpec((1, _O), lambda b: (0, 0)),
            pl.BlockSpec((1, _O), lambda b: (0, 0)),
        ],
        out_specs=pl.BlockSpec((_S, _N, _O), lambda b: (b, 0, 0)),
        out_shape=jax.ShapeDtypeStruct((_B, _N, _O), jnp.float32),
        compiler_params=pltpu.CompilerParams(
            dimension_semantics=("parallel",)),
    )(x, W, a_src.reshape(1, _O), a_dst.reshape(1, _O), bias.reshape(1, _O))
    return out
